# Initial kernel scaffold; baseline (speedup 1.0000x reference)
import jax
import jax.numpy as jnp
from jax.experimental import pallas as pl


def kernel(user_table, item_table, layer_weights, edge_index):
    raise NotImplementedError("write your pallas kernel here")


# speedup vs baseline: 10.3309x; 10.3309x over previous
import functools

import jax
import jax.numpy as jnp
from jax import lax
from jax.experimental import pallas as pl
from jax.experimental.pallas import tpu as pltpu
from jax.experimental.pallas import tpu_sc as plsc

NUM_USERS = 25000
NUM_ITEMS = 25000
D = 64
NUM_LAYERS = 3
N_EDGES = 800000
NN = NUM_USERS + NUM_ITEMS

NC = 2
NS = 16
HALF = NN // NC
TRASH = HALF
ACC_ROWS = HALF + 88
ZR = ACC_ROWS // NS
ECH = 128
GB = 7
NG = 56
NCH = NG * GB
EPT = NCH * ECH
NPAD = NS * EPT
PADCOL = 1 << 20

_mesh = plsc.VectorSubcoreMesh(
    core_axis_name="c", subcore_axis_name="s", num_cores=NC, num_subcores=NS
)


@functools.partial(
    pl.kernel,
    out_type=(
        jax.ShapeDtypeStruct((NC, ACC_ROWS, 16), jnp.float32),
        jax.ShapeDtypeStruct((NC, NS, NCH, ECH), jnp.int32),
    ),
    mesh=_mesh,
    compiler_params=pltpu.CompilerParams(use_tc_tiling_on_sc=False),
    scratch_types=[
        pltpu.VMEM((GB, ECH), jnp.int32),
        pltpu.VMEM((GB, ECH), jnp.int32),
        pltpu.VMEM((ECH, 16), jnp.float32),
        pltpu.VMEM_SHARED((ACC_ROWS, 16), jnp.float32),
    ],
)
def _sc_degree(col_hbm, zeros16_hbm, degp_hbm, colfix_hbm, cbuf, fbuf, ones_v, acc_sh):
    c = lax.axis_index("c")
    s = lax.axis_index("s")
    sc_off = c * HALF

    def fill_ones(i, _):
        ones_v[i, :] = jnp.ones((16,), jnp.float32)
        return 0

    lax.fori_loop(0, ECH, fill_ones, 0)
    pltpu.sync_copy(zeros16_hbm.at[pl.ds(s * ZR, ZR)], acc_sh.at[pl.ds(s * ZR, ZR)])
    plsc.subcore_barrier()

    def group(gd, _):
        pltpu.sync_copy(col_hbm.at[s, pl.ds(gd * GB, GB)], cbuf)

        def fix_chunk(j, _):
            def fix_group(g, _):
                cc = cbuf[j, pl.ds(g * 16, 16)]
                loc = cc - sc_off
                valid = (loc >= 0) & (loc < HALF)
                fbuf[j, pl.ds(g * 16, 16)] = jnp.where(valid, loc, TRASH)
                return 0

            lax.fori_loop(0, ECH // 16, fix_group, 0)
            return 0

        lax.fori_loop(0, GB, fix_chunk, 0)
        pltpu.sync_copy(fbuf, colfix_hbm.at[c, s, pl.ds(gd * GB, GB)])

        def add_chunk(j, _):
            pltpu.sync_copy(ones_v, acc_sh.at[fbuf.at[j]], add=True)
            return 0

        lax.fori_loop(0, GB, add_chunk, 0)
        return 0

    lax.fori_loop(0, NG, group, 0)
    plsc.subcore_barrier()
    pltpu.sync_copy(acc_sh.at[pl.ds(s * ZR, ZR)], degp_hbm.at[c, pl.ds(s * ZR, ZR)])


@functools.partial(
    pl.kernel,
    out_type=jax.ShapeDtypeStruct((NC, ACC_ROWS, D), jnp.float32),
    mesh=_mesh,
    compiler_params=pltpu.CompilerParams(use_tc_tiling_on_sc=False),
    scratch_types=[
        pltpu.VMEM((2 * GB, ECH), jnp.int32),
        pltpu.VMEM((2 * GB, ECH), jnp.int32),
        pltpu.VMEM((ECH, D), jnp.float32),
        pltpu.VMEM((ECH, D), jnp.float32),
        pltpu.VMEM_SHARED((ACC_ROWS, D), jnp.float32),
        pltpu.SemaphoreType.DMA,
        pltpu.SemaphoreType.DMA,
        pltpu.SemaphoreType.DMA,
        pltpu.SemaphoreType.DMA,
    ],
)
def _sc_propagate(t_hbm, icomb_hbm, zeros_hbm, out_hbm,
                  ibuf_a, ibuf_b, rows0, rows1, acc_sh, sem_a, sem_b, sem0, sem1):
    c = lax.axis_index("c")
    s = lax.axis_index("s")
    pltpu.sync_copy(zeros_hbm.at[pl.ds(s * ZR, ZR)], acc_sh.at[pl.ds(s * ZR, ZR)])
    plsc.subcore_barrier()

    pltpu.make_async_copy(icomb_hbm.at[c, s, 0], ibuf_a, sem_a).start()
    pltpu.make_async_copy(icomb_hbm.at[c, s, 1], ibuf_b, sem_b).start()

    rbufs = (rows0, rows1)
    gsems = (sem0, sem1)

    def process_group(g, ibuf, isem, nextg):
        pltpu.make_async_copy(icomb_hbm.at[c, s, g], ibuf, isem).wait()
        pltpu.make_async_copy(t_hbm.at[ibuf.at[0]], rows0, sem0).start()
        pltpu.make_async_copy(t_hbm.at[ibuf.at[1]], rows1, sem1).start()
        for k in range(GB):
            b = k % 2
            pltpu.make_async_copy(t_hbm.at[ibuf.at[k]], rbufs[b], gsems[b]).wait()
            pltpu.sync_copy(rbufs[b], acc_sh.at[ibuf.at[GB + k]], add=True)
            if k + 2 < GB:
                pltpu.make_async_copy(
                    t_hbm.at[ibuf.at[k + 2]], rbufs[b], gsems[b]).start()

        @pl.when(nextg < NG)
        def _():
            pltpu.make_async_copy(icomb_hbm.at[c, s, nextg], ibuf, isem).start()

    def body(g2, _):
        g = 2 * g2
        process_group(g, ibuf_a, sem_a, g + 2)
        process_group(g + 1, ibuf_b, sem_b, g + 3)
        return 0

    lax.fori_loop(0, NG // 2, body, 0)
    plsc.subcore_barrier()
    pltpu.sync_copy(acc_sh.at[pl.ds(s * ZR, ZR)], out_hbm.at[c, pl.ds(s * ZR, ZR)])


_RB = 1000


def _prep_body(deg_ref, x_ref, t_ref, dinv_ref, deginv_ref):
    deg = deg_ref[:, 0:1] + 1.0
    dinv = lax.rsqrt(deg)
    deginv = dinv * dinv
    x = x_ref[:]
    t_ref[:] = x * dinv
    dinv_ref[:] = jnp.broadcast_to(dinv, (_RB, D))
    deginv_ref[:] = jnp.broadcast_to(deginv, (_RB, D))


def _tc_prep(deg_counts, x0):
    return pl.pallas_call(
        _prep_body,
        grid=(NN // _RB,),
        in_specs=[
            pl.BlockSpec((_RB, 16), lambda i: (i, 0)),
            pl.BlockSpec((_RB, D), lambda i: (i, 0)),
        ],
        out_specs=[
            pl.BlockSpec((_RB, D), lambda i: (i, 0)),
            pl.BlockSpec((_RB, D), lambda i: (i, 0)),
            pl.BlockSpec((_RB, D), lambda i: (i, 0)),
        ],
        out_shape=[
            jax.ShapeDtypeStruct((NN, D), jnp.float32),
            jax.ShapeDtypeStruct((NN, D), jnp.float32),
            jax.ShapeDtypeStruct((NN, D), jnp.float32),
        ],
    )(deg_counts, x0)


def _combine_body(s_ref, xp_ref, dinv_ref, deginv_ref, x_ref, t_ref):
    x = dinv_ref[:] * s_ref[:] + deginv_ref[:] * xp_ref[:]
    x_ref[:] = x
    t_ref[:] = dinv_ref[:] * x


def _tc_combine(s, xprev, dinv_b, deginv_b):
    w = 2 * D
    spec = pl.BlockSpec((_RB, w), lambda i: (i, 0))
    return pl.pallas_call(
        _combine_body,
        grid=(NN // 2 // _RB,),
        in_specs=[spec, spec, spec, spec],
        out_specs=[spec, spec],
        out_shape=[
            jax.ShapeDtypeStruct((NN // 2, w), jnp.float32),
            jax.ShapeDtypeStruct((NN // 2, w), jnp.float32),
        ],
    )(s, xprev, dinv_b, deginv_b)


def _final_body(w_ref, x0_ref, x1_ref, x2_ref, x3_ref, out_ref):
    acc = (w_ref[0, 0] * x0_ref[:] + w_ref[0, 1] * x1_ref[:]
           + w_ref[0, 2] * x2_ref[:] + w_ref[0, 3] * x3_ref[:])
    nrm = jnp.sqrt(jnp.sum(acc * acc, axis=1, keepdims=True))
    out_ref[:] = acc / jnp.maximum(nrm, 1e-12)


def _tc_final(wvec, x0, x1, x2, x3):
    spec = pl.BlockSpec((_RB, D), lambda i: (i, 0))
    return pl.pallas_call(
        _final_body,
        grid=(NN // _RB,),
        in_specs=[pl.BlockSpec(memory_space=pltpu.SMEM), spec, spec, spec, spec],
        out_specs=pl.BlockSpec((_RB, D), lambda i: (i, 0)),
        out_shape=jax.ShapeDtypeStruct((NN, D), jnp.float32),
    )(wvec, x0, x1, x2, x3)


def kernel(user_table, item_table, layer_weights, edge_index):
    x0 = jnp.concatenate([user_table, item_table], axis=0)
    ei = edge_index.astype(jnp.int32)
    pad = NPAD - N_EDGES
    row_pad = jnp.reshape(
        jnp.concatenate([ei[0], jnp.zeros((pad,), jnp.int32)]), (NS, NCH, ECH))
    col_pad = jnp.reshape(
        jnp.concatenate([ei[1], jnp.full((pad,), PADCOL, jnp.int32)]), (NS, NCH, ECH))
    zeros16 = jnp.zeros((ACC_ROWS, 16), jnp.float32)
    zeros64 = jnp.zeros((ACC_ROWS, D), jnp.float32)

    degp, colfix = _sc_degree(col_pad, zeros16)
    deg_counts = jnp.reshape(degp[:, :HALF, :], (NN, 16))

    rowg = jnp.broadcast_to(
        jnp.reshape(row_pad, (1, NS, NG, 1, GB, ECH)), (NC, NS, NG, 1, GB, ECH))
    colg = jnp.reshape(colfix, (NC, NS, NG, 1, GB, ECH))
    icomb = jnp.reshape(
        jnp.concatenate([rowg, colg], axis=3), (NC, NS, NG, 2 * GB, ECH))

    t, dinv_b, deginv_b = _tc_prep(deg_counts, x0)

    def half_lane(a):
        return jnp.reshape(a, (NN // 2, 2 * D))

    dinv_h, deginv_h = half_lane(dinv_b), half_lane(deginv_b)
    embeddings = [x0]
    xprev = x0
    for _ in range(NUM_LAYERS):
        s_parts = _sc_propagate(t, icomb, zeros64)
        s_full = jnp.reshape(s_parts[:, :HALF, :], (NN, D))
        x_h, t_h = _tc_combine(half_lane(s_full), half_lane(xprev), dinv_h, deginv_h)
        xprev = jnp.reshape(x_h, (NN, D))
        t = jnp.reshape(t_h, (NN, D))
        embeddings.append(xprev)

    wvec = jnp.reshape(layer_weights.astype(jnp.float32), (1, NUM_LAYERS + 1))
    final = _tc_final(wvec, *embeddings)
    return (final[:NUM_USERS], final[NUM_USERS:])


# speedup vs baseline: 19.7060x; 1.9075x over previous
import functools

import jax
import jax.numpy as jnp
from jax import lax
from jax.experimental import pallas as pl
from jax.experimental.pallas import tpu as pltpu
from jax.experimental.pallas import tpu_sc as plsc

NUM_USERS = 25000
NUM_ITEMS = 25000
D = 64
DH = D // 2
NUM_LAYERS = 3
N_EDGES = 800000
NN = NUM_USERS + NUM_ITEMS

NC = 2
NS = 16
TRASH = NN
ACC_ROWS = NN + 16
ZR = ACC_ROWS // NS
ECH = 128
GB = 7
NG = 56
NCH = NG * GB
EPT = NCH * ECH
NPAD = NS * EPT
NGD = 28
NCHD = NGD * GB

_mesh = plsc.VectorSubcoreMesh(
    core_axis_name="c", subcore_axis_name="s", num_cores=NC, num_subcores=NS
)


@functools.partial(
    pl.kernel,
    out_type=jax.ShapeDtypeStruct((NC, ACC_ROWS, 16), jnp.float32),
    mesh=_mesh,
    compiler_params=pltpu.CompilerParams(use_tc_tiling_on_sc=False),
    scratch_types=[
        pltpu.VMEM((GB, ECH), jnp.int32),
        pltpu.VMEM((GB, ECH), jnp.int32),
        pltpu.VMEM((ECH, 16), jnp.float32),
        pltpu.VMEM_SHARED((ACC_ROWS, 16), jnp.float32),
        pltpu.SemaphoreType.DMA,
        pltpu.SemaphoreType.DMA,
    ],
)
def _sc_degree(col_hbm, zeros16_hbm, degp_hbm, cbuf_a, cbuf_b, ones_v, acc_sh,
               sem_a, sem_b):
    c = lax.axis_index("c")
    s = lax.axis_index("s")

    def fill_ones(i, _):
        ones_v[i, :] = jnp.ones((16,), jnp.float32)
        return 0

    lax.fori_loop(0, ECH, fill_ones, 0)
    pltpu.sync_copy(zeros16_hbm.at[pl.ds(s * ZR, ZR)], acc_sh.at[pl.ds(s * ZR, ZR)])
    plsc.subcore_barrier()

    pltpu.make_async_copy(col_hbm.at[c, s, pl.ds(0, GB)], cbuf_a, sem_a).start()
    pltpu.make_async_copy(col_hbm.at[c, s, pl.ds(GB, GB)], cbuf_b, sem_b).start()

    def process_group(g, cbuf, sem, nextg):
        pltpu.make_async_copy(col_hbm.at[c, s, pl.ds(g * GB, GB)], cbuf, sem).wait()

        def add_chunk(j, _):
            pltpu.sync_copy(ones_v, acc_sh.at[cbuf.at[j]], add=True)
            return 0

        lax.fori_loop(0, GB, add_chunk, 0)

        @pl.when(nextg < NGD)
        def _():
            pltpu.make_async_copy(
                col_hbm.at[c, s, pl.ds(nextg * GB, GB)], cbuf, sem).start()

    def body(g2, _):
        g = 2 * g2
        process_group(g, cbuf_a, sem_a, g + 2)
        process_group(g + 1, cbuf_b, sem_b, g + 3)
        return 0

    lax.fori_loop(0, NGD // 2, body, 0)
    plsc.subcore_barrier()
    pltpu.sync_copy(acc_sh.at[pl.ds(s * ZR, ZR)], degp_hbm.at[c, pl.ds(s * ZR, ZR)])


@functools.partial(
    pl.kernel,
    out_type=jax.ShapeDtypeStruct((NC, ACC_ROWS, DH), jnp.float32),
    mesh=_mesh,
    compiler_params=pltpu.CompilerParams(use_tc_tiling_on_sc=False),
    scratch_types=[
        pltpu.VMEM((GB, ECH), jnp.int32),
        pltpu.VMEM((GB, ECH), jnp.int32),
        pltpu.VMEM((GB, ECH), jnp.int32),
        pltpu.VMEM((GB, ECH), jnp.int32),
        pltpu.VMEM((ECH, DH), jnp.float32),
        pltpu.VMEM((ECH, DH), jnp.float32),
        pltpu.VMEM_SHARED((ACC_ROWS, DH), jnp.float32),
        pltpu.SemaphoreType.DMA,
        pltpu.SemaphoreType.DMA,
        pltpu.SemaphoreType.DMA,
        pltpu.SemaphoreType.DMA,
        pltpu.SemaphoreType.DMA,
        pltpu.SemaphoreType.DMA,
    ],
)
def _sc_propagate(t_hbm, row_hbm, col_hbm, zeros_hbm, out_hbm,
                  rb_a, cb_a, rb_b, cb_b, rows0, rows1, acc_sh,
                  semr_a, semc_a, semr_b, semc_b, sem0, sem1):
    c = lax.axis_index("c")
    s = lax.axis_index("s")
    pltpu.sync_copy(zeros_hbm.at[pl.ds(s * ZR, ZR)], acc_sh.at[pl.ds(s * ZR, ZR)])
    plsc.subcore_barrier()

    pltpu.make_async_copy(row_hbm.at[c, s, pl.ds(0, GB)], rb_a, semr_a).start()
    pltpu.make_async_copy(col_hbm.at[s, pl.ds(0, GB)], cb_a, semc_a).start()
    pltpu.make_async_copy(row_hbm.at[c, s, pl.ds(GB, GB)], rb_b, semr_b).start()
    pltpu.make_async_copy(col_hbm.at[s, pl.ds(GB, GB)], cb_b, semc_b).start()

    rbufs = (rows0, rows1)
    gsems = (sem0, sem1)

    def process_group(g, rb, cb, semr, semc, nextg):
        pltpu.make_async_copy(row_hbm.at[c, s, pl.ds(g * GB, GB)], rb, semr).wait()
        pltpu.make_async_copy(col_hbm.at[s, pl.ds(g * GB, GB)], cb, semc).wait()
        pltpu.make_async_copy(t_hbm.at[rb.at[0]], rows0, sem0).start()
        pltpu.make_async_copy(t_hbm.at[rb.at[1]], rows1, sem1).start()
        for k in range(GB):
            b = k % 2
            pltpu.make_async_copy(t_hbm.at[rb.at[k]], rbufs[b], gsems[b]).wait()
            pltpu.sync_copy(rbufs[b], acc_sh.at[cb.at[k]], add=True)
            if k + 2 < GB:
                pltpu.make_async_copy(
                    t_hbm.at[rb.at[k + 2]], rbufs[b], gsems[b]).start()

        @pl.when(nextg < NG)
        def _():
            pltpu.make_async_copy(
                row_hbm.at[c, s, pl.ds(nextg * GB, GB)], rb, semr).start()
            pltpu.make_async_copy(
                col_hbm.at[s, pl.ds(nextg * GB, GB)], cb, semc).start()

    def body(g2, _):
        g = 2 * g2
        process_group(g, rb_a, cb_a, semr_a, semc_a, g + 2)
        process_group(g + 1, rb_b, cb_b, semr_b, semc_b, g + 3)
        return 0

    lax.fori_loop(0, NG // 2, body, 0)
    plsc.subcore_barrier()
    pltpu.sync_copy(acc_sh.at[pl.ds(s * ZR, ZR)], out_hbm.at[c, pl.ds(s * ZR, ZR)])


_RB = 1000


def _prep_body(degp_ref, x_ref, t_ref, dinv_ref, deginv_ref):
    deg = degp_ref[0, :, 0:1] + degp_ref[1, :, 0:1] + 1.0
    dinv = lax.rsqrt(deg)
    deginv = dinv * dinv
    x = x_ref[:]
    t_ref[0] = x[:, :DH] * dinv
    t_ref[1] = x[:, DH:] * dinv
    dinv_ref[:] = jnp.broadcast_to(dinv, (_RB, D))
    deginv_ref[:] = jnp.broadcast_to(deginv, (_RB, D))


def _tc_prep(degp, x0):
    return pl.pallas_call(
        _prep_body,
        grid=(NN // _RB,),
        in_specs=[
            pl.BlockSpec((NC, _RB, 16), lambda i: (0, i, 0)),
            pl.BlockSpec((_RB, D), lambda i: (i, 0)),
        ],
        out_specs=[
            pl.BlockSpec((NC, _RB, DH), lambda i: (0, i, 0)),
            pl.BlockSpec((_RB, D), lambda i: (i, 0)),
            pl.BlockSpec((_RB, D), lambda i: (i, 0)),
        ],
        out_shape=[
            jax.ShapeDtypeStruct((NC, NN, DH), jnp.float32),
            jax.ShapeDtypeStruct((NN, D), jnp.float32),
            jax.ShapeDtypeStruct((NN, D), jnp.float32),
        ],
    )(degp, x0)


def _combine_body(s_ref, xp_ref, dinv_ref, deginv_ref, x_ref, t_ref):
    da = dinv_ref[:, :DH]
    db = dinv_ref[:, DH:]
    xa = da * s_ref[0] + deginv_ref[:, :DH] * xp_ref[:, :DH]
    xb = db * s_ref[1] + deginv_ref[:, DH:] * xp_ref[:, DH:]
    x_ref[:, :DH] = xa
    x_ref[:, DH:] = xb
    t_ref[0] = da * xa
    t_ref[1] = db * xb


def _tc_combine(s_parts, xprev, dinv_b, deginv_b):
    spec64 = pl.BlockSpec((_RB, D), lambda i: (i, 0))
    return pl.pallas_call(
        _combine_body,
        grid=(NN // _RB,),
        in_specs=[
            pl.BlockSpec((NC, _RB, DH), lambda i: (0, i, 0)),
            spec64, spec64, spec64,
        ],
        out_specs=[
            spec64,
            pl.BlockSpec((NC, _RB, DH), lambda i: (0, i, 0)),
        ],
        out_shape=[
            jax.ShapeDtypeStruct((NN, D), jnp.float32),
            jax.ShapeDtypeStruct((NC, NN, DH), jnp.float32),
        ],
    )(s_parts, xprev, dinv_b, deginv_b)


def _final_body(w_ref, x0_ref, x1_ref, x2_ref, x3_ref, out_ref):
    acc = (w_ref[0, 0] * x0_ref[:] + w_ref[0, 1] * x1_ref[:]
           + w_ref[0, 2] * x2_ref[:] + w_ref[0, 3] * x3_ref[:])
    nrm = jnp.sqrt(jnp.sum(acc * acc, axis=1, keepdims=True))
    out_ref[:] = acc / jnp.maximum(nrm, 1e-12)


def _tc_final(wvec, x0, x1, x2, x3):
    spec = pl.BlockSpec((_RB, D), lambda i: (i, 0))
    return pl.pallas_call(
        _final_body,
        grid=(NN // _RB,),
        in_specs=[pl.BlockSpec(memory_space=pltpu.SMEM), spec, spec, spec, spec],
        out_specs=pl.BlockSpec((_RB, D), lambda i: (i, 0)),
        out_shape=jax.ShapeDtypeStruct((NN, D), jnp.float32),
    )(wvec, x0, x1, x2, x3)


def kernel(user_table, item_table, layer_weights, edge_index):
    x0 = jnp.concatenate([user_table, item_table], axis=0)
    ei = edge_index.astype(jnp.int32)
    pad = NPAD - N_EDGES
    row1d = jnp.concatenate([ei[0], jnp.zeros((pad,), jnp.int32)])
    col1d = jnp.concatenate([ei[1], jnp.full((pad,), TRASH, jnp.int32)])
    rows_off = jnp.reshape(
        jnp.stack([row1d, row1d + NN]), (NC, NS, NCH, ECH))
    col_prop = jnp.reshape(col1d, (NS, NCH, ECH))
    col_deg = jnp.reshape(col1d, (NC, NS, NCHD, ECH))
    zeros16 = jnp.zeros((ACC_ROWS, 16), jnp.float32)
    zeros32 = jnp.zeros((ACC_ROWS, DH), jnp.float32)

    degp = _sc_degree(col_deg, zeros16)
    t3, dinv_b, deginv_b = _tc_prep(degp, x0)
    t = jnp.reshape(t3, (NC * NN, DH))

    embeddings = [x0]
    xprev = x0
    for _ in range(NUM_LAYERS):
        s_parts = _sc_propagate(t, rows_off, col_prop, zeros32)
        xprev, t3 = _tc_combine(s_parts, xprev, dinv_b, deginv_b)
        t = jnp.reshape(t3, (NC * NN, DH))
        embeddings.append(xprev)

    wvec = jnp.reshape(layer_weights.astype(jnp.float32), (1, NUM_LAYERS + 1))
    final = _tc_final(wvec, *embeddings)
    return (final[:NUM_USERS], final[NUM_USERS:])


# speedup vs baseline: 20.7568x; 1.0533x over previous
import functools

import jax
import jax.numpy as jnp
from jax import lax
from jax.experimental import pallas as pl
from jax.experimental.pallas import tpu as pltpu
from jax.experimental.pallas import tpu_sc as plsc

NUM_USERS = 25000
NUM_ITEMS = 25000
D = 64
DH = D // 2
NUM_LAYERS = 3
N_EDGES = 800000
NN = NUM_USERS + NUM_ITEMS

NC = 2
NS = 16
TRASH = NN
ACC_ROWS = 50176
ZR = ACC_ROWS // NS
CB = 56
NCB = ZR // CB
ECH = 128
GB = 7
NG = 56
NCH = NG * GB
EPT = NCH * ECH
NPAD = NS * EPT
NGD = 28
NCHD = NGD * GB

_mesh = plsc.VectorSubcoreMesh(
    core_axis_name="c", subcore_axis_name="s", num_cores=NC, num_subcores=NS
)


@functools.partial(
    pl.kernel,
    out_type=jax.ShapeDtypeStruct((NC, ACC_ROWS, 16), jnp.float32),
    mesh=_mesh,
    compiler_params=pltpu.CompilerParams(use_tc_tiling_on_sc=False),
    scratch_types=[
        pltpu.VMEM((GB, ECH), jnp.int32),
        pltpu.VMEM((GB, ECH), jnp.int32),
        pltpu.VMEM((ECH, 16), jnp.float32),
        pltpu.VMEM_SHARED((ACC_ROWS, 16), jnp.float32),
        pltpu.SemaphoreType.DMA,
        pltpu.SemaphoreType.DMA,
    ],
)
def _sc_degree(col_hbm, zeros16_hbm, degp_hbm, cbuf_a, cbuf_b, ones_v, acc_sh,
               sem_a, sem_b):
    c = lax.axis_index("c")
    s = lax.axis_index("s")

    def fill_ones(i, _):
        ones_v[i, :] = jnp.ones((16,), jnp.float32)
        return 0

    lax.fori_loop(0, ECH, fill_ones, 0)
    pltpu.sync_copy(zeros16_hbm.at[pl.ds(s * ZR, ZR)], acc_sh.at[pl.ds(s * ZR, ZR)])
    plsc.subcore_barrier()

    pltpu.make_async_copy(col_hbm.at[c, s, pl.ds(0, GB)], cbuf_a, sem_a).start()
    pltpu.make_async_copy(col_hbm.at[c, s, pl.ds(GB, GB)], cbuf_b, sem_b).start()

    def process_group(g, cbuf, sem, nextg):
        pltpu.make_async_copy(col_hbm.at[c, s, pl.ds(g * GB, GB)], cbuf, sem).wait()

        def add_chunk(j, _):
            pltpu.sync_copy(ones_v, acc_sh.at[cbuf.at[j]], add=True)
            return 0

        lax.fori_loop(0, GB, add_chunk, 0)

        @pl.when(nextg < NGD)
        def _():
            pltpu.make_async_copy(
                col_hbm.at[c, s, pl.ds(nextg * GB, GB)], cbuf, sem).start()

    def body(g2, _):
        g = 2 * g2
        process_group(g, cbuf_a, sem_a, g + 2)
        process_group(g + 1, cbuf_b, sem_b, g + 3)
        return 0

    lax.fori_loop(0, NGD // 2, body, 0)
    plsc.subcore_barrier()
    pltpu.sync_copy(acc_sh.at[pl.ds(s * ZR, ZR)], degp_hbm.at[c, pl.ds(s * ZR, ZR)])


@functools.partial(
    pl.kernel,
    out_type=(
        jax.ShapeDtypeStruct((NUM_LAYERS, NC, ACC_ROWS, DH), jnp.float32),
        jax.ShapeDtypeStruct((NC * ACC_ROWS, DH), jnp.float32),
    ),
    mesh=_mesh,
    compiler_params=pltpu.CompilerParams(use_tc_tiling_on_sc=False),
    scratch_types=[
        pltpu.VMEM((GB, ECH), jnp.int32),
        pltpu.VMEM((GB, ECH), jnp.int32),
        pltpu.VMEM((GB, ECH), jnp.int32),
        pltpu.VMEM((GB, ECH), jnp.int32),
        pltpu.VMEM((ECH, DH), jnp.float32),
        pltpu.VMEM((ECH, DH), jnp.float32),
        pltpu.VMEM((CB, 16), jnp.float32),
        pltpu.VMEM((CB, 16), jnp.float32),
        pltpu.VMEM((CB, DH), jnp.float32),
        pltpu.VMEM((CB, DH), jnp.float32),
        pltpu.VMEM((CB, DH), jnp.float32),
        pltpu.VMEM((CB, DH), jnp.float32),
        pltpu.VMEM_SHARED((ACC_ROWS, DH), jnp.float32),
        pltpu.SemaphoreType.DMA,
        pltpu.SemaphoreType.DMA,
        pltpu.SemaphoreType.DMA,
        pltpu.SemaphoreType.DMA,
        pltpu.SemaphoreType.DMA,
        pltpu.SemaphoreType.DMA,
        pltpu.SemaphoreType.DMA,
        pltpu.SemaphoreType.DMA,
        pltpu.SemaphoreType.DMA,
        pltpu.SemaphoreType.DMA,
        pltpu.SemaphoreType.DMA,
        pltpu.SemaphoreType.DMA,
    ],
)
def _sc_layers(t0_hbm, row_hbm, col_hbm, dinv_hbm, xout_hbm, twork_hbm,
               rb_a, cb_a, rb_b, cb_b, rows0, rows1, db_a, db_b,
               xs_a, xs_b, ts_a, ts_b, acc_sh,
               semr_a, semc_a, semr_b, semc_b, sem0, sem1,
               semd_a, semd_b, semx_a, semx_b, semt_a, semt_b):
    c = lax.axis_index("c")
    s = lax.axis_index("s")
    base = s * ZR

    pltpu.sync_copy(t0_hbm.at[pl.ds(c * ACC_ROWS + base, ZR)],
                    acc_sh.at[pl.ds(base, ZR)])
    plsc.subcore_barrier()

    rbufs = (rows0, rows1)
    gsems = (sem0, sem1)
    dbufs = (db_a, db_b)
    dsems = (semd_a, semd_b)
    xbufs = (xs_a, xs_b)
    xsems = (semx_a, semx_b)
    tbufs = (ts_a, ts_b)
    tsems = (semt_a, semt_b)

    def propagate(t_hbm):
        def process_group(g, rb, cb, semr, semc, nextg):
            pltpu.make_async_copy(
                row_hbm.at[c, s, pl.ds(g * GB, GB)], rb, semr).wait()
            pltpu.make_async_copy(
                col_hbm.at[s, pl.ds(g * GB, GB)], cb, semc).wait()
            pltpu.make_async_copy(t_hbm.at[rb.at[0]], rows0, sem0).start()
            pltpu.make_async_copy(t_hbm.at[rb.at[1]], rows1, sem1).start()
            for k in range(GB):
                b = k % 2
                pltpu.make_async_copy(t_hbm.at[rb.at[k]], rbufs[b], gsems[b]).wait()
                pltpu.sync_copy(rbufs[b], acc_sh.at[cb.at[k]], add=True)
                if k + 2 < GB:
                    pltpu.make_async_copy(
                        t_hbm.at[rb.at[k + 2]], rbufs[b], gsems[b]).start()

            @pl.when(nextg < NG)
            def _():
                pltpu.make_async_copy(
                    row_hbm.at[c, s, pl.ds(nextg * GB, GB)], rb, semr).start()
                pltpu.make_async_copy(
                    col_hbm.at[s, pl.ds(nextg * GB, GB)], cb, semc).start()

        pltpu.make_async_copy(row_hbm.at[c, s, pl.ds(0, GB)], rb_a, semr_a).start()
        pltpu.make_async_copy(col_hbm.at[s, pl.ds(0, GB)], cb_a, semc_a).start()
        pltpu.make_async_copy(row_hbm.at[c, s, pl.ds(GB, GB)], rb_b, semr_b).start()
        pltpu.make_async_copy(col_hbm.at[s, pl.ds(GB, GB)], cb_b, semc_b).start()

        def body(g2, _):
            g = 2 * g2
            process_group(g, rb_a, cb_a, semr_a, semc_a, g + 2)
            process_group(g + 1, rb_b, cb_b, semr_b, semc_b, g + 3)
            return 0

        lax.fori_loop(0, NG // 2, body, 0)

    def combine(r, last):
        def start_reads(n, b):
            pltpu.make_async_copy(
                acc_sh.at[pl.ds(base + n * CB, CB)], rbufs[b].at[pl.ds(0, CB)],
                gsems[b]).start()
            pltpu.make_async_copy(
                dinv_hbm.at[pl.ds(base + n * CB, CB)], dbufs[b], dsems[b]).start()

        def process_chunk(n, b, first, nextn, do_next=True):
            pltpu.make_async_copy(
                acc_sh.at[pl.ds(base + n * CB, CB)], rbufs[b].at[pl.ds(0, CB)],
                gsems[b]).wait()
            pltpu.make_async_copy(
                dinv_hbm.at[pl.ds(base + n * CB, CB)], dbufs[b], dsems[b]).wait()
            if not first:
                pltpu.make_async_copy(
                    xbufs[b], xout_hbm.at[r, c, pl.ds(base + (n - 2) * CB, CB)],
                    xsems[b]).wait()
                if not last:
                    pltpu.make_async_copy(
                        tbufs[b],
                        twork_hbm.at[pl.ds(c * ACC_ROWS + base + (n - 2) * CB, CB)],
                        tsems[b]).wait()

            def row_body(i, _):
                dv = dbufs[b][i, :]
                for g in range(2):
                    a = rbufs[b][i, pl.ds(g * 16, 16)]
                    x = dv * a
                    xbufs[b][i, pl.ds(g * 16, 16)] = x
                    if not last:
                        tbufs[b][i, pl.ds(g * 16, 16)] = dv * x
                return 0

            lax.fori_loop(0, CB, row_body, 0)

            if not last:
                pltpu.sync_copy(tbufs[b], acc_sh.at[pl.ds(base + n * CB, CB)])
            pltpu.make_async_copy(
                xbufs[b], xout_hbm.at[r, c, pl.ds(base + n * CB, CB)],
                xsems[b]).start()
            if not last:
                pltpu.make_async_copy(
                    tbufs[b],
                    twork_hbm.at[pl.ds(c * ACC_ROWS + base + n * CB, CB)],
                    tsems[b]).start()
            if do_next:
                start_reads(nextn, b)

        start_reads(0, 0)
        start_reads(1, 1)
        process_chunk(0, 0, True, 2)
        process_chunk(1, 1, True, 3)

        def body(m, _):
            n = 2 * m
            process_chunk(n, 0, False, n + 2)
            process_chunk(n + 1, 1, False, n + 3)
            return 0

        lax.fori_loop(1, NCB // 2 - 1, body, 0)
        process_chunk(NCB - 2, 0, False, NCB, do_next=False)
        process_chunk(NCB - 1, 1, False, NCB + 1, do_next=False)
        for b in range(2):
            pltpu.make_async_copy(
                xbufs[b], xout_hbm.at[r, c, pl.ds(base + (NCB - 2 + b) * CB, CB)],
                xsems[b]).wait()
            if not last:
                pltpu.make_async_copy(
                    tbufs[b],
                    twork_hbm.at[pl.ds(c * ACC_ROWS + base + (NCB - 2 + b) * CB, CB)],
                    tsems[b]).wait()

    for r in range(NUM_LAYERS):
        propagate(t0_hbm if r == 0 else twork_hbm)
        plsc.subcore_barrier()
        combine(r, r == NUM_LAYERS - 1)
        plsc.subcore_barrier()


_RB = 1568


def _prep_body(degp_ref, x_ref, t_ref, dinv_ref):
    deg = degp_ref[0, :, 0:1] + degp_ref[1, :, 0:1] + 1.0
    dinv = lax.rsqrt(deg)
    x = x_ref[:]
    t_ref[0] = x[:, :DH] * dinv
    t_ref[1] = x[:, DH:] * dinv
    dinv_ref[:] = jnp.broadcast_to(dinv, (_RB, 16))


def _tc_prep(degp, x0p):
    return pl.pallas_call(
        _prep_body,
        grid=(ACC_ROWS // _RB,),
        in_specs=[
            pl.BlockSpec((NC, _RB, 16), lambda i: (0, i, 0)),
            pl.BlockSpec((_RB, D), lambda i: (i, 0)),
        ],
        out_specs=[
            pl.BlockSpec((NC, _RB, DH), lambda i: (0, i, 0)),
            pl.BlockSpec((_RB, 16), lambda i: (i, 0)),
        ],
        out_shape=[
            jax.ShapeDtypeStruct((NC, ACC_ROWS, DH), jnp.float32),
            jax.ShapeDtypeStruct((ACC_ROWS, 16), jnp.float32),
        ],
    )(degp, x0p)


_RF = 1000


def _final_body(w_ref, x0_ref, x1_ref, x2_ref, x3_ref, out_ref):
    acc_a = w_ref[0, 0] * x0_ref[:, :DH]
    acc_b = w_ref[0, 0] * x0_ref[:, DH:]
    for i, xr in enumerate((x1_ref, x2_ref, x3_ref)):
        acc_a = acc_a + w_ref[0, i + 1] * xr[0]
        acc_b = acc_b + w_ref[0, i + 1] * xr[1]
    nrm2 = (jnp.sum(acc_a * acc_a, axis=1, keepdims=True)
            + jnp.sum(acc_b * acc_b, axis=1, keepdims=True))
    scale = 1.0 / jnp.maximum(jnp.sqrt(nrm2), 1e-12)
    out_ref[:, :DH] = acc_a * scale
    out_ref[:, DH:] = acc_b * scale


def _tc_final(wvec, x0, x1, x2, x3):
    spec_split = pl.BlockSpec((NC, _RF, DH), lambda i: (0, i, 0))
    return pl.pallas_call(
        _final_body,
        grid=(NN // _RF,),
        in_specs=[
            pl.BlockSpec(memory_space=pltpu.SMEM),
            pl.BlockSpec((_RF, D), lambda i: (i, 0)),
            spec_split, spec_split, spec_split,
        ],
        out_specs=pl.BlockSpec((_RF, D), lambda i: (i, 0)),
        out_shape=jax.ShapeDtypeStruct((NN, D), jnp.float32),
    )(wvec, x0, x1, x2, x3)


def kernel(user_table, item_table, layer_weights, edge_index):
    x0p = jnp.concatenate(
        [user_table, item_table,
         jnp.zeros((ACC_ROWS - NN, D), jnp.float32)], axis=0)
    ei = edge_index.astype(jnp.int32)
    pad = NPAD - N_EDGES
    row1d = jnp.concatenate([ei[0], jnp.zeros((pad,), jnp.int32)])
    col1d = jnp.concatenate([ei[1], jnp.full((pad,), TRASH, jnp.int32)])
    rows_off = jnp.reshape(
        jnp.stack([row1d, row1d + ACC_ROWS]), (NC, NS, NCH, ECH))
    col_prop = jnp.reshape(col1d, (NS, NCH, ECH))
    col_deg = jnp.reshape(col1d, (NC, NS, NCHD, ECH))
    zeros16 = jnp.zeros((ACC_ROWS, 16), jnp.float32)

    degp = _sc_degree(col_deg, zeros16)
    t3, dinv16 = _tc_prep(degp, x0p)
    t0 = jnp.reshape(t3, (NC * ACC_ROWS, DH))

    xout, _ = _sc_layers(t0, rows_off, col_prop, dinv16)

    wvec = jnp.reshape(layer_weights.astype(jnp.float32), (1, NUM_LAYERS + 1))
    final = _tc_final(wvec, x0p[:NN], xout[0], xout[1], xout[2])
    return (final[:NUM_USERS], final[NUM_USERS:])
